# Initial kernel scaffold; baseline (speedup 1.0000x reference)
import jax
import jax.numpy as jnp
from jax.experimental import pallas as pl


def kernel(x, edge_index, W):
    raise NotImplementedError("write your pallas kernel here")


# speedup vs baseline: 3.9843x; 3.9843x over previous
import functools

import jax
import jax.numpy as jnp
from jax import lax
from jax.experimental import pallas as pl
from jax.experimental.pallas import tpu as pltpu
from jax.experimental.pallas import tpu_sc as plsc

B = 4
FIN = 128
FOUT = 128
V = 10000
E = 320000
K = 3

NC = 2
NS = 16
NW = NC * NS
EPT = E // NW
EPS = E // NS
IB = 128
SPIB = 64
NFULL = EPS // SPIB
REM = EPS - NFULL * SPIB
SLAB = 80
NSLAB = V // SLAB
CPC = FIN
NCHUNK = (B * FIN) // CPC

_mesh = plsc.VectorSubcoreMesh(core_axis_name="c", subcore_axis_name="s")


def _zero16():
    return jnp.zeros((16,), jnp.float32)


DEG_NFULL = EPT // IB
DEG_REM = EPT - DEG_NFULL * IB
DSLAB = 1000


@functools.partial(
    pl.kernel,
    mesh=_mesh,
    out_type=jax.ShapeDtypeStruct((NC * V,), jnp.float32),
    scratch_types=[
        pltpu.VMEM_SHARED((V,), jnp.float32),
        pltpu.VMEM((DSLAB,), jnp.float32),
        pltpu.VMEM((DSLAB,), jnp.float32),
        pltpu.VMEM((IB,), jnp.float32),
        pltpu.VMEM((IB,), jnp.int32),
        pltpu.VMEM((DEG_REM,), jnp.int32),
    ],
)
def _deg_kernel(rows_hbm, degp_hbm, acc1, zb1, db1, obuf, rbuf, rbuf_r):
    cid = lax.axis_index("c")
    sid = lax.axis_index("s")
    wid = cid * NS + sid
    ebase = wid * EPT

    def zfill(i, _):
        zb1[pl.ds(i * 16, 16)] = _zero16()
        return 0

    lax.fori_loop(0, DSLAB // 16, zfill, 0)
    for q in range(IB // 16):
        obuf[pl.ds(q * 16, 16)] = jnp.ones((16,), jnp.float32)

    @pl.when(sid < V // DSLAB)
    def _():
        pltpu.sync_copy(zb1, acc1.at[pl.ds(sid * DSLAB, DSLAB)])

    plsc.subcore_barrier()

    def blk(j, _):
        off = ebase + j * IB
        pltpu.sync_copy(rows_hbm.at[pl.ds(off, IB)], rbuf)
        pltpu.sync_copy(obuf, acc1.at[rbuf], add=True)
        return 0

    lax.fori_loop(0, DEG_NFULL, blk, 0)
    off = ebase + DEG_NFULL * IB
    pltpu.sync_copy(rows_hbm.at[pl.ds(off, DEG_REM)], rbuf_r)
    pltpu.sync_copy(obuf.at[pl.ds(0, DEG_REM)], acc1.at[rbuf_r], add=True)

    plsc.subcore_barrier()

    @pl.when(sid < V // DSLAB)
    def _():
        pltpu.sync_copy(acc1.at[pl.ds(sid * DSLAB, DSLAB)], db1)
        pltpu.sync_copy(db1, degp_hbm.at[pl.ds(cid * V + sid * DSLAB, DSLAB)])


def _prep_body(x_ref, degp_ref, u0_ref, dis_ref, dis2n_ref):
    deg = jnp.sum(degp_ref[...], axis=0)
    dis = lax.rsqrt(jnp.maximum(deg, 1.0))
    dis_ref[...] = dis
    dis2n_ref[...] = jnp.broadcast_to((-(dis * dis))[:, None], dis2n_ref.shape)
    xt = jnp.transpose(x_ref[0], (1, 0))
    u0_ref[0] = xt * dis[:, None]


def _prep(x, degp):
    return pl.pallas_call(
        _prep_body,
        grid=(B,),
        in_specs=[
            pl.BlockSpec((1, FIN, V), lambda b: (b, 0, 0)),
            pl.BlockSpec((NC, V), lambda b: (0, 0)),
        ],
        out_specs=[
            pl.BlockSpec((1, V, FIN), lambda b: (b, 0, 0)),
            pl.BlockSpec((V,), lambda b: (0,)),
            pl.BlockSpec((V, FIN), lambda b: (0, 0)),
        ],
        out_shape=[
            jax.ShapeDtypeStruct((B, V, FIN), jnp.float32),
            jax.ShapeDtypeStruct((V,), jnp.float32),
            jax.ShapeDtypeStruct((V, FIN), jnp.float32),
        ],
    )(x, degp)


@functools.partial(
    pl.kernel,
    mesh=_mesh,
    out_type=[
        jax.ShapeDtypeStruct((NCHUNK * V, FIN), jnp.float32),
        jax.ShapeDtypeStruct((NCHUNK * V, FIN), jnp.float32),
        jax.ShapeDtypeStruct((NCHUNK * V, FIN), jnp.float32),
    ],
    scratch_types=[
        pltpu.VMEM_SHARED((V, FIN), jnp.float32),
        pltpu.VMEM((SLAB, FIN), jnp.float32),
        pltpu.VMEM((SLAB, FIN), jnp.float32),
        pltpu.VMEM((SLAB, FIN), jnp.float32),
        pltpu.VMEM((SPIB,), jnp.int32),
        pltpu.VMEM((SPIB,), jnp.int32),
        pltpu.VMEM((REM,), jnp.int32),
        pltpu.VMEM((REM,), jnp.int32),
        pltpu.VMEM((SPIB, FIN), jnp.float32),
        pltpu.VMEM((REM, FIN), jnp.float32),
        pltpu.SemaphoreType.DMA,
    ],
)
def _spmm_kernel(u0_hbm, rows_hbm, cols_hbm, dis2n_hbm,
                 g1_hbm, g2_hbm, u1_hbm,
                 acc, zbuf, dbuf, sbuf,
                 cbuf, rbuf, cbuf_r, rbuf_r, gbuf, gbuf_r, sem):
    cid = lax.axis_index("c")
    sid = lax.axis_index("s")
    ebase = sid * EPS

    def zb(i, _):
        zbuf[i, pl.ds(0 * 16, 16)] = _zero16()
        zbuf[i, pl.ds(1 * 16, 16)] = _zero16()
        zbuf[i, pl.ds(2 * 16, 16)] = _zero16()
        zbuf[i, pl.ds(3 * 16, 16)] = _zero16()
        zbuf[i, pl.ds(4 * 16, 16)] = _zero16()
        zbuf[i, pl.ds(5 * 16, 16)] = _zero16()
        zbuf[i, pl.ds(6 * 16, 16)] = _zero16()
        zbuf[i, pl.ds(7 * 16, 16)] = _zero16()
        return 0

    lax.fori_loop(0, SLAB, zb, 0)

    def my_slabs(fn):
        for t in range((NSLAB + NS - 1) // NS):
            k = sid + NS * t
            if (t + 1) * NS <= NSLAB:
                fn(k)
            else:
                @pl.when(k < NSLAB)
                def _():
                    fn(k)

    def zero_slab(k):
        pltpu.sync_copy(zbuf, acc.at[pl.ds(k * SLAB, SLAB)])

    my_slabs(zero_slab)
    plsc.subcore_barrier()

    def edge_pass(src_hbm, coff):
        def blk(j, _):
            off = ebase + j * SPIB
            pltpu.sync_copy(cols_hbm.at[pl.ds(off, SPIB)], cbuf)
            pltpu.sync_copy(rows_hbm.at[pl.ds(off, SPIB)], rbuf)
            for q in range(SPIB // 16):
                sl = pl.ds(q * 16, 16)
                cbuf[sl] = cbuf[sl] + coff
            pltpu.async_copy(src_hbm.at[cbuf], gbuf, sem).wait()
            pltpu.sync_copy(gbuf, acc.at[rbuf], add=True)
            return 0

        lax.fori_loop(0, NFULL, blk, 0)
        off = ebase + NFULL * SPIB
        pltpu.sync_copy(cols_hbm.at[pl.ds(off, REM)], cbuf_r)
        pltpu.sync_copy(rows_hbm.at[pl.ds(off, REM)], rbuf_r)
        for q in range(REM // 16):
            sl = pl.ds(q * 16, 16)
            cbuf_r[sl] = cbuf_r[sl] + coff
        pltpu.async_copy(src_hbm.at[cbuf_r], gbuf_r, sem).wait()
        pltpu.sync_copy(gbuf_r, acc.at[rbuf_r], add=True)

    for ci in range(NCHUNK // NC):
        c = cid * (NCHUNK // NC) + ci
        coff = c * V

        edge_pass(u0_hbm, coff)
        plsc.subcore_barrier()

        def drain1(k):
            r0 = k * SLAB
            pltpu.sync_copy(acc.at[pl.ds(r0, SLAB)], dbuf)
            pltpu.sync_copy(zbuf, acc.at[pl.ds(r0, SLAB)])
            pltpu.sync_copy(dbuf, g1_hbm.at[pl.ds(coff + r0, SLAB)])
            pltpu.sync_copy(dis2n_hbm.at[pl.ds(r0, SLAB)], sbuf)

            def rowscale(r, _):
                for q in range(FIN // 16):
                    sl = pl.ds(q * 16, 16)
                    dbuf[r, sl] = dbuf[r, sl] * sbuf[r, sl]
                return 0

            lax.fori_loop(0, SLAB, rowscale, 0)
            pltpu.sync_copy(dbuf, u1_hbm.at[pl.ds(coff + r0, SLAB)])

        my_slabs(drain1)
        plsc.subcore_barrier()

        edge_pass(u1_hbm, coff)
        plsc.subcore_barrier()

        def drain2(k):
            r0 = k * SLAB
            pltpu.sync_copy(acc.at[pl.ds(r0, SLAB)], dbuf)
            pltpu.sync_copy(zbuf, acc.at[pl.ds(r0, SLAB)])
            pltpu.sync_copy(dbuf, g2_hbm.at[pl.ds(coff + r0, SLAB)])

        my_slabs(drain2)
        plsc.subcore_barrier()


def _epi_body(x_ref, g1_ref, g2_ref, dis_ref, wr_ref, out_ref):
    w0 = wr_ref[0]
    w1 = wr_ref[1]
    w2 = wr_ref[2]
    xb = x_ref[0]
    g1 = g1_ref[0]
    g2 = g2_ref[0]
    dn0 = (((1,), (0,)), ((), ()))
    dn1 = (((1,), (1,)), ((), ()))
    m0 = lax.dot_general(w0 - w2, xb, dn0, preferred_element_type=jnp.float32)
    m1 = lax.dot_general(w1, g1, dn1, preferred_element_type=jnp.float32)
    m2 = lax.dot_general(w2, g2, dn1, preferred_element_type=jnp.float32)
    out_ref[0] = m0 - (m1 + 2.0 * m2) * dis_ref[...][None, :]


def _epilogue(x, g1, g2, dis, wr):
    vb = V
    return pl.pallas_call(
        _epi_body,
        grid=(B, V // vb),
        in_specs=[
            pl.BlockSpec((1, FIN, vb), lambda b, i: (b, 0, i)),
            pl.BlockSpec((1, vb, FIN), lambda b, i: (b, i, 0)),
            pl.BlockSpec((1, vb, FIN), lambda b, i: (b, i, 0)),
            pl.BlockSpec((vb,), lambda b, i: (i,)),
            pl.BlockSpec((K, FOUT, FIN), lambda b, i: (0, 0, 0)),
        ],
        out_specs=pl.BlockSpec((1, FOUT, vb), lambda b, i: (b, 0, i)),
        out_shape=jax.ShapeDtypeStruct((B, FOUT, V), jnp.float32),
    )(x, g1, g2, dis, wr)


def kernel(x, edge_index, W):
    rows = edge_index[0]
    cols = edge_index[1]
    degp = _deg_kernel(rows)
    u0, dis, dis2n = _prep(x, degp.reshape(NC, V))
    g1, g2, _u1 = _spmm_kernel(u0.reshape(NCHUNK * V, FIN), rows, cols, dis2n)
    wr = jnp.transpose(W.reshape(FOUT, FIN, K), (2, 0, 1))
    out = _epilogue(x, g1.reshape(B, V, FIN), g2.reshape(B, V, FIN), dis, wr)
    return out


# speedup vs baseline: 4.5365x; 1.1386x over previous
import functools

import jax
import jax.numpy as jnp
from jax import lax
from jax.experimental import pallas as pl
from jax.experimental.pallas import tpu as pltpu
from jax.experimental.pallas import tpu_sc as plsc

B = 4
FIN = 128
FOUT = 128
V = 10000
E = 320000
K = 3

NC = 2
NS = 16
NW = NC * NS
EPT = E // NW
EPS = E // NS
IB = 128
SPIB = 64
NBLK_T = 320
NSTG_BLK = 16
NSTAGE = NBLK_T // NSTG_BLK
EPAD = NS * NBLK_T * SPIB
VP = 10240
SLAB = 80
NSLAB = V // SLAB
CPC = FIN
NCHUNK = (B * FIN) // CPC

_mesh = plsc.VectorSubcoreMesh(core_axis_name="c", subcore_axis_name="s")


def _zero16():
    return jnp.zeros((16,), jnp.float32)


DEG_NFULL = EPT // IB
DEG_REM = EPT - DEG_NFULL * IB
DSLAB = 1000


@functools.partial(
    pl.kernel,
    mesh=_mesh,
    out_type=jax.ShapeDtypeStruct((NC * V,), jnp.float32),
    scratch_types=[
        pltpu.VMEM_SHARED((V,), jnp.float32),
        pltpu.VMEM((DSLAB,), jnp.float32),
        pltpu.VMEM((DSLAB,), jnp.float32),
        pltpu.VMEM((IB,), jnp.float32),
        pltpu.VMEM((IB,), jnp.int32),
        pltpu.VMEM((DEG_REM,), jnp.int32),
    ],
)
def _deg_kernel(rows_hbm, degp_hbm, acc1, zb1, db1, obuf, rbuf, rbuf_r):
    cid = lax.axis_index("c")
    sid = lax.axis_index("s")
    wid = cid * NS + sid
    ebase = wid * EPT

    def zfill(i, _):
        zb1[pl.ds(i * 16, 16)] = _zero16()
        return 0

    lax.fori_loop(0, DSLAB // 16, zfill, 0)
    for q in range(IB // 16):
        obuf[pl.ds(q * 16, 16)] = jnp.ones((16,), jnp.float32)

    @pl.when(sid < V // DSLAB)
    def _():
        pltpu.sync_copy(zb1, acc1.at[pl.ds(sid * DSLAB, DSLAB)])

    plsc.subcore_barrier()

    def blk(j, _):
        off = ebase + j * IB
        pltpu.sync_copy(rows_hbm.at[pl.ds(off, IB)], rbuf)
        pltpu.sync_copy(obuf, acc1.at[rbuf], add=True)
        return 0

    lax.fori_loop(0, DEG_NFULL, blk, 0)
    off = ebase + DEG_NFULL * IB
    pltpu.sync_copy(rows_hbm.at[pl.ds(off, DEG_REM)], rbuf_r)
    pltpu.sync_copy(obuf.at[pl.ds(0, DEG_REM)], acc1.at[rbuf_r], add=True)

    plsc.subcore_barrier()

    @pl.when(sid < V // DSLAB)
    def _():
        pltpu.sync_copy(acc1.at[pl.ds(sid * DSLAB, DSLAB)], db1)
        pltpu.sync_copy(db1, degp_hbm.at[pl.ds(cid * V + sid * DSLAB, DSLAB)])


def _prep_body(x_ref, degp_ref, u0_ref, dis_ref, dis2n_ref):
    deg = jnp.sum(degp_ref[...], axis=0)
    dis = lax.rsqrt(jnp.maximum(deg, 1.0))
    dis_ref[...] = dis
    dis2n_ref[...] = jnp.broadcast_to((-(dis * dis))[:, None], dis2n_ref.shape)
    xt = jnp.transpose(x_ref[0], (1, 0))
    u0_ref[0] = xt * dis[:, None]


def _prep(x, degp):
    return pl.pallas_call(
        _prep_body,
        grid=(B,),
        in_specs=[
            pl.BlockSpec((1, FIN, V), lambda b: (b, 0, 0)),
            pl.BlockSpec((NC, V), lambda b: (0, 0)),
        ],
        out_specs=[
            pl.BlockSpec((1, V, FIN), lambda b: (b, 0, 0)),
            pl.BlockSpec((V,), lambda b: (0,)),
            pl.BlockSpec((V, FIN), lambda b: (0, 0)),
        ],
        out_shape=[
            jax.ShapeDtypeStruct((B, V, FIN), jnp.float32),
            jax.ShapeDtypeStruct((V,), jnp.float32),
            jax.ShapeDtypeStruct((V, FIN), jnp.float32),
        ],
    )(x, degp)


@functools.partial(
    pl.kernel,
    mesh=_mesh,
    out_type=[
        jax.ShapeDtypeStruct((NCHUNK * V, FIN), jnp.float32),
        jax.ShapeDtypeStruct((NCHUNK * V, FIN), jnp.float32),
        jax.ShapeDtypeStruct((NCHUNK * V, FIN), jnp.float32),
    ],
    scratch_types=[
        pltpu.VMEM_SHARED((VP, FIN), jnp.float32),
        pltpu.VMEM((SLAB, FIN), jnp.float32),
        pltpu.VMEM((SLAB, FIN), jnp.float32),
        pltpu.VMEM((2, NSTG_BLK, SPIB), jnp.int32),
        pltpu.VMEM((2, NSTG_BLK, SPIB), jnp.int32),
        pltpu.VMEM((2, SPIB, FIN), jnp.float32),
        pltpu.SemaphoreType.DMA,
        pltpu.SemaphoreType.DMA,
        pltpu.SemaphoreType.DMA,
        pltpu.SemaphoreType.DMA,
    ],
)
def _spmm_kernel(u0_hbm, r2d_hbm, c2d_hbm, dis2n_hbm,
                 g1_hbm, g2_hbm, u1_hbm,
                 acc, dbuf, sbuf, cstg, rstg, gbuf,
                 sem_c, sem_r, sem_g0, sem_g1):
    cid = lax.axis_index("c")
    sid = lax.axis_index("s")
    bb = sid * NBLK_T

    def zero_dbuf():
        def zrow(r, _):
            for q in range(FIN // 16):
                dbuf[r, pl.ds(q * 16, 16)] = _zero16()
            return 0

        lax.fori_loop(0, SLAB, zrow, 0)

    def my_slabs(fn):
        for t in range((NSLAB + NS - 1) // NS):
            k = sid + NS * t
            if (t + 1) * NS <= NSLAB:
                fn(k)
            else:
                @pl.when(k < NSLAB)
                def _():
                    fn(k)

    zero_dbuf()
    my_slabs(lambda k: pltpu.sync_copy(dbuf, acc.at[pl.ds(k * SLAB, SLAB)]))
    plsc.subcore_barrier()

    def edge_pass(src_hbm, coff):
        def stage_dma(s):
            p = lax.rem(s, 2)
            r0 = bb + s * NSTG_BLK
            pltpu.async_copy(c2d_hbm.at[pl.ds(r0, NSTG_BLK)], cstg.at[p], sem_c)
            pltpu.async_copy(r2d_hbm.at[pl.ds(r0, NSTG_BLK)], rstg.at[p], sem_r)

        def sbody(s, _):
            p = lax.rem(s, 2)
            r0 = bb + s * NSTG_BLK
            pltpu.make_async_copy(
                c2d_hbm.at[pl.ds(r0, NSTG_BLK)], cstg.at[p], sem_c).wait()
            pltpu.make_async_copy(
                r2d_hbm.at[pl.ds(r0, NSTG_BLK)], rstg.at[p], sem_r).wait()

            @pl.when(s < NSTAGE - 1)
            def _():
                stage_dma(s + 1)

            for j in range(NSTG_BLK):
                for q in range(SPIB // 16):
                    sl = pl.ds(q * 16, 16)
                    cstg[p, j, sl] = cstg[p, j, sl] + coff

            h = [None, None]
            h[0] = pltpu.async_copy(src_hbm.at[cstg.at[p, 0]], gbuf.at[0],
                                    sem_g0)
            for j in range(NSTG_BLK):
                jp = j % 2
                if j + 1 < NSTG_BLK:
                    h[1 - jp] = pltpu.async_copy(
                        src_hbm.at[cstg.at[p, j + 1]], gbuf.at[1 - jp],
                        sem_g1 if (j + 1) % 2 else sem_g0)
                h[jp].wait()
                pltpu.sync_copy(gbuf.at[jp], acc.at[rstg.at[p, j]], add=True)
            return 0

        stage_dma(0)
        lax.fori_loop(0, NSTAGE, sbody, 0)

    for ci in range(NCHUNK // NC):
        c = cid * (NCHUNK // NC) + ci
        coff = c * V

        edge_pass(u0_hbm, coff)
        plsc.subcore_barrier()

        def drain1(k):
            r0 = k * SLAB
            pltpu.sync_copy(acc.at[pl.ds(r0, SLAB)], dbuf)
            pltpu.sync_copy(dbuf, g1_hbm.at[pl.ds(coff + r0, SLAB)])
            pltpu.sync_copy(dis2n_hbm.at[pl.ds(r0, SLAB)], sbuf)

            def rowscale(r, _):
                for q in range(FIN // 16):
                    sl = pl.ds(q * 16, 16)
                    dbuf[r, sl] = dbuf[r, sl] * sbuf[r, sl]
                return 0

            lax.fori_loop(0, SLAB, rowscale, 0)
            pltpu.sync_copy(dbuf, u1_hbm.at[pl.ds(coff + r0, SLAB)])
            zero_dbuf()
            pltpu.sync_copy(dbuf, acc.at[pl.ds(r0, SLAB)])

        my_slabs(drain1)
        plsc.subcore_barrier()

        edge_pass(u1_hbm, coff)
        plsc.subcore_barrier()

        def drain2(k):
            r0 = k * SLAB
            pltpu.sync_copy(acc.at[pl.ds(r0, SLAB)], dbuf)
            pltpu.sync_copy(dbuf, g2_hbm.at[pl.ds(coff + r0, SLAB)])
            zero_dbuf()
            pltpu.sync_copy(dbuf, acc.at[pl.ds(r0, SLAB)])

        my_slabs(drain2)
        plsc.subcore_barrier()


def _epi_body(x_ref, g1_ref, g2_ref, dis_ref, wr_ref, out_ref):
    w0 = wr_ref[0]
    w1 = wr_ref[1]
    w2 = wr_ref[2]
    xb = x_ref[0]
    g1 = g1_ref[0]
    g2 = g2_ref[0]
    dn0 = (((1,), (0,)), ((), ()))
    dn1 = (((1,), (1,)), ((), ()))
    m0 = lax.dot_general(w0 - w2, xb, dn0, preferred_element_type=jnp.float32)
    m1 = lax.dot_general(w1, g1, dn1, preferred_element_type=jnp.float32)
    m2 = lax.dot_general(w2, g2, dn1, preferred_element_type=jnp.float32)
    out_ref[0] = m0 - (m1 + 2.0 * m2) * dis_ref[...][None, :]


def _epilogue(x, g1, g2, dis, wr):
    vb = V
    return pl.pallas_call(
        _epi_body,
        grid=(B, V // vb),
        in_specs=[
            pl.BlockSpec((1, FIN, vb), lambda b, i: (b, 0, i)),
            pl.BlockSpec((1, vb, FIN), lambda b, i: (b, i, 0)),
            pl.BlockSpec((1, vb, FIN), lambda b, i: (b, i, 0)),
            pl.BlockSpec((vb,), lambda b, i: (i,)),
            pl.BlockSpec((K, FOUT, FIN), lambda b, i: (0, 0, 0)),
        ],
        out_specs=pl.BlockSpec((1, FOUT, vb), lambda b, i: (b, 0, i)),
        out_shape=jax.ShapeDtypeStruct((B, FOUT, V), jnp.float32),
    )(x, g1, g2, dis, wr)


def kernel(x, edge_index, W):
    rows = edge_index[0]
    cols = edge_index[1]
    degp = _deg_kernel(rows)
    u0, dis, dis2n = _prep(x, degp.reshape(NC, V))
    npad = EPAD - E
    c2d = jnp.concatenate(
        [cols, jnp.zeros((npad,), jnp.int32)]).reshape(-1, SPIB)
    r2d = jnp.concatenate(
        [rows, V + (jnp.arange(npad, dtype=jnp.int32) % (VP - V))]
    ).reshape(-1, SPIB)
    g1, g2, _u1 = _spmm_kernel(u0.reshape(NCHUNK * V, FIN), r2d, c2d, dis2n)
    wr = jnp.transpose(W.reshape(FOUT, FIN, K), (2, 0, 1))
    out = _epilogue(x, g1.reshape(B, V, FIN), g2.reshape(B, V, FIN), dis, wr)
    return out
